# Initial kernel scaffold; baseline (speedup 1.0000x reference)
import jax
import jax.numpy as jnp
from jax.experimental import pallas as pl


def kernel(x, train_edge_index, train_pos_edge_index, negative_edge_index, W1_l, W1_r, b1, W2_l, W2_r, b2, W_lin, b_lin):
    raise NotImplementedError("write your pallas kernel here")


# speedup vs baseline: 4.6660x; 4.6660x over previous
import functools

import jax
import jax.numpy as jnp
from jax import lax
from jax.experimental import pallas as pl
from jax.experimental.pallas import tpu as pltpu
from jax.experimental.pallas import tpu_sc as plsc

N = 100000
E = 1600000
EP = 500000

NC, NS, L = 2, 16, 16
BN = 512
NPAD = 100352
ROWS_PER_TILE = NPAD // NS
C = 128
EPAD = 1601536
HALF_E = EPAD // 2
CHUNKS1 = HALF_E // (NS * C)
CHUNKS2 = EPAD // (NS * C)
ED = 2 * EP
EDPAD = 1003520
DCHUNKS = EDPAD // (NC * NS * C)

_MESH = plsc.VectorSubcoreMesh(
    core_axis_name="c", subcore_axis_name="s", num_cores=NC, num_subcores=NS)
_SC_PARAMS = pltpu.CompilerParams(
    use_tc_tiling_on_sc=False, needs_layout_passes=False)


def _zero_rows8(ref):
    iota = jnp.arange(L, dtype=jnp.int32)
    z = jnp.zeros((L,), jnp.float32)
    for g in range(C // L):
        er = iota + (g * L)
        for col in range(8):
            plsc.store_scatter(ref, [er, jnp.full((L,), col, jnp.int32)], z)


@functools.partial(
    pl.kernel,
    out_type=jax.ShapeDtypeStruct((2 * NPAD, 8), jnp.float32),
    mesh=_MESH,
    compiler_params=_SC_PARAMS,
    scratch_types=dict(
        sidx_v=pltpu.VMEM((C,), jnp.int32),
        didx_v=pltpu.VMEM((C,), jnp.int32),
        rows_v=pltpu.VMEM((C, 8), jnp.float32),
        zrows_v=pltpu.VMEM((C, 8), jnp.float32),
        fbuf_v=pltpu.VMEM((C, 8), jnp.float32),
        acc_sh=pltpu.VMEM_SHARED((NPAD, 8), jnp.float32),
        sem=pltpu.SemaphoreType.DMA,
    ),
)
def _sc_agg_scalar(src_hbm, dst_hbm, xr_hbm, out_hbm,
                   sidx_v, didx_v, rows_v, zrows_v, fbuf_v, acc_sh, sem):
    c = lax.axis_index("c")
    s = lax.axis_index("s")

    _zero_rows8(zrows_v)

    def zbody(j, carry):
        pltpu.sync_copy(zrows_v,
                        acc_sh.at[pl.ds(s * ROWS_PER_TILE + j * C, C), :])
        return carry
    lax.fori_loop(0, ROWS_PER_TILE // C, zbody, 0)
    plsc.subcore_barrier()

    def body(j, carry):
        base = c * HALF_E + (s * CHUNKS1 + j) * C
        pltpu.sync_copy(src_hbm.at[pl.ds(base, C)], sidx_v)
        pltpu.async_copy(xr_hbm.at[sidx_v], rows_v, sem).wait()
        pltpu.sync_copy(dst_hbm.at[pl.ds(base, C)], didx_v)
        pltpu.sync_copy(rows_v, acc_sh.at[didx_v], add=True)
        return carry
    lax.fori_loop(0, CHUNKS1, body, 0)
    plsc.subcore_barrier()

    def fbody(j, carry):
        row = s * ROWS_PER_TILE + j * C
        pltpu.sync_copy(acc_sh.at[pl.ds(row, C), :], fbuf_v)
        pltpu.sync_copy(fbuf_v, out_hbm.at[pl.ds(c * NPAD + row, C), :])
        return carry
    lax.fori_loop(0, ROWS_PER_TILE // C, fbody, 0)


@functools.partial(
    pl.kernel,
    out_type=jax.ShapeDtypeStruct((4 * NPAD, 16), jnp.float32),
    mesh=_MESH,
    compiler_params=_SC_PARAMS,
    scratch_types=dict(
        sidx_v=pltpu.VMEM((C,), jnp.int32),
        didx_v=pltpu.VMEM((C,), jnp.int32),
        adj_v=pltpu.VMEM((C,), jnp.int32),
        rows_v=pltpu.VMEM((C, 16), jnp.float32),
        zrows_v=pltpu.VMEM((C, 16), jnp.float32),
        fbuf_v=pltpu.VMEM((C, 16), jnp.float32),
        acc_sh=pltpu.VMEM_SHARED((NPAD, 16), jnp.float32),
        sem=pltpu.SemaphoreType.DMA,
    ),
)
def _sc_agg64(src_hbm, dst_hbm, g1_hbm, s2_hbm,
              sidx_v, didx_v, adj_v, rows_v, zrows_v, fbuf_v, acc_sh, sem):
    c = lax.axis_index("c")
    s = lax.axis_index("s")

    for r in range(C):
        zrows_v[r, :] = jnp.zeros((16,), jnp.float32)

    for rnd in range(2):
        fboff = (c * 2 + rnd) * NPAD

        def zbody(j, carry):
            pltpu.sync_copy(
                zrows_v, acc_sh.at[pl.ds(s * ROWS_PER_TILE + j * C, C), :])
            return carry
        lax.fori_loop(0, ROWS_PER_TILE // C, zbody, 0)
        plsc.subcore_barrier()

        def body(j, carry):
            base = (s * CHUNKS2 + j) * C
            pltpu.sync_copy(src_hbm.at[pl.ds(base, C)], sidx_v)
            for k in range(C // L):
                sl = pl.ds(k * L, L)
                adj_v[sl] = sidx_v[sl] + fboff
            pltpu.async_copy(g1_hbm.at[adj_v], rows_v, sem).wait()
            pltpu.sync_copy(dst_hbm.at[pl.ds(base, C)], didx_v)
            pltpu.sync_copy(rows_v, acc_sh.at[didx_v], add=True)
            return carry
        lax.fori_loop(0, CHUNKS2, body, 0)
        plsc.subcore_barrier()

        def fbody(j, carry):
            row = s * ROWS_PER_TILE + j * C
            pltpu.sync_copy(acc_sh.at[pl.ds(row, C), :], fbuf_v)
            pltpu.sync_copy(fbuf_v, s2_hbm.at[pl.ds(fboff + row, C), :])
            return carry
        lax.fori_loop(0, ROWS_PER_TILE // C, fbody, 0)
        plsc.subcore_barrier()


@functools.partial(
    pl.kernel,
    out_type=jax.ShapeDtypeStruct((4 * EDPAD,), jnp.float32),
    mesh=_MESH,
    compiler_params=_SC_PARAMS,
    scratch_types=dict(
        i0_v=pltpu.VMEM((C,), jnp.int32),
        i1_v=pltpu.VMEM((C,), jnp.int32),
        r0_v=pltpu.VMEM((C, 16), jnp.float32),
        r1_v=pltpu.VMEM((C, 16), jnp.float32),
        o_v=pltpu.VMEM((4 * C,), jnp.float32),
        sem=pltpu.SemaphoreType.DMA,
    ),
)
def _sc_decode(e0_hbm, e1_hbm, z_hbm, out_hbm,
               i0_v, i1_v, r0_v, r1_v, o_v, sem):
    c = lax.axis_index("c")
    s = lax.axis_index("s")
    wid = s * NC + c
    iota = jnp.arange(L, dtype=jnp.int32)

    def body(j, carry):
        base = (wid * DCHUNKS + j) * C
        pltpu.sync_copy(e0_hbm.at[pl.ds(base, C)], i0_v)
        pltpu.sync_copy(e1_hbm.at[pl.ds(base, C)], i1_v)
        pltpu.async_copy(z_hbm.at[i0_v], r0_v, sem).wait()
        pltpu.async_copy(z_hbm.at[i1_v], r1_v, sem).wait()
        for g in range(C // L):
            er = iota + (g * L)
            lg = []
            for cls in range(4):
                csp = jnp.full((L,), cls, jnp.int32)
                v0 = plsc.load_gather(r0_v, [er, csp])
                v1 = plsc.load_gather(r1_v, [er, csp])
                lg.append(v0 * v1)
            m = jnp.maximum(jnp.maximum(lg[0], lg[1]),
                            jnp.maximum(lg[2], lg[3]))
            ex = [jnp.exp(v - m) for v in lg]
            tot = (ex[0] + ex[1]) + (ex[2] + ex[3])
            for cls in range(4):
                plsc.store_scatter(o_v, [er * 4 + cls], ex[cls] / tot)
        pltpu.sync_copy(o_v, out_hbm.at[pl.ds(base * 4, 4 * C)])
        return carry
    lax.fori_loop(0, DCHUNKS, body, 0)


def _tc_dense1_body(p_ref, x_ref, w1l_ref, w1r_ref, b1_ref,
                    w2l_ref, w2r_ref, b2_ref, g1_ref, r1_ref, rec_ref):
    s1 = p_ref[0, :, 0:1] + p_ref[1, :, 0:1]
    cnt = p_ref[0, :, 1:2] + p_ref[1, :, 1:2]
    rec = 1.0 / jnp.maximum(cnt, 1.0)
    rec_ref[...] = rec
    a = s1 * rec
    h1 = jnp.maximum(
        a * w1l_ref[...] + x_ref[...] * w1r_ref[...] + b1_ref[...], 0.0)
    g = jnp.dot(h1, w2l_ref[...], preferred_element_type=jnp.float32)
    r = jnp.dot(h1, w2r_ref[...], preferred_element_type=jnp.float32)
    r = r + b2_ref[...]
    for fb in range(4):
        g1_ref[fb, :, :] = g[:, fb * 16:(fb + 1) * 16]
        r1_ref[fb, :, :] = r[:, fb * 16:(fb + 1) * 16]


_tc_dense1 = pl.pallas_call(
    _tc_dense1_body,
    grid=(NPAD // BN,),
    in_specs=[
        pl.BlockSpec((2, BN, 8), lambda i: (0, i, 0)),
        pl.BlockSpec((BN, 1), lambda i: (i, 0)),
        pl.BlockSpec((1, 128), lambda i: (0, 0)),
        pl.BlockSpec((1, 128), lambda i: (0, 0)),
        pl.BlockSpec((1, 128), lambda i: (0, 0)),
        pl.BlockSpec((128, 64), lambda i: (0, 0)),
        pl.BlockSpec((128, 64), lambda i: (0, 0)),
        pl.BlockSpec((1, 64), lambda i: (0, 0)),
    ],
    out_specs=[
        pl.BlockSpec((4, BN, 16), lambda i: (0, i, 0)),
        pl.BlockSpec((4, BN, 16), lambda i: (0, i, 0)),
        pl.BlockSpec((BN, 1), lambda i: (i, 0)),
    ],
    out_shape=[
        jax.ShapeDtypeStruct((4, NPAD, 16), jnp.float32),
        jax.ShapeDtypeStruct((4, NPAD, 16), jnp.float32),
        jax.ShapeDtypeStruct((NPAD, 1), jnp.float32),
    ],
)


def _tc_dense2_body(s2_ref, r1_ref, rec_ref, wlin_ref, blin_ref, z_ref):
    rec = rec_ref[...]
    acc = jnp.zeros((BN, 4), jnp.float32)
    for fb in range(4):
        u = jnp.maximum(s2_ref[fb, :, :] * rec + r1_ref[fb, :, :], 0.0)
        acc = acc + jnp.dot(u, wlin_ref[fb, :, :],
                            preferred_element_type=jnp.float32)
    z_ref[...] = jnp.concatenate(
        [acc + blin_ref[...], jnp.zeros((BN, 12), jnp.float32)], axis=1)


_tc_dense2 = pl.pallas_call(
    _tc_dense2_body,
    grid=(NPAD // BN,),
    in_specs=[
        pl.BlockSpec((4, BN, 16), lambda i: (0, i, 0)),
        pl.BlockSpec((4, BN, 16), lambda i: (0, i, 0)),
        pl.BlockSpec((BN, 1), lambda i: (i, 0)),
        pl.BlockSpec((4, 16, 4), lambda i: (0, 0, 0)),
        pl.BlockSpec((1, 4), lambda i: (0, 0)),
    ],
    out_specs=pl.BlockSpec((BN, 16), lambda i: (i, 0)),
    out_shape=jax.ShapeDtypeStruct((NPAD, 16), jnp.float32),
)


def kernel(x, train_edge_index, train_pos_edge_index, negative_edge_index,
           W1_l, W1_r, b1, W2_l, W2_r, b2, W_lin, b_lin):
    src = train_edge_index[0].astype(jnp.int32)
    dst = train_edge_index[1].astype(jnp.int32)
    epad = jnp.int32(N) + (jnp.arange(EPAD - E, dtype=jnp.int32) % 256)
    srcp = jnp.concatenate([src, epad])
    dstp = jnp.concatenate([dst, epad])
    xp = jnp.pad(x[:, 0], (0, NPAD - N))
    xr = jnp.concatenate(
        [xp[:, None], jnp.ones((NPAD, 1), jnp.float32),
         jnp.zeros((NPAD, 6), jnp.float32)], axis=1)

    parts = _sc_agg_scalar(srcp, dstp, xr)

    g1b, r1b, rec = _tc_dense1(
        parts.reshape(2, NPAD, 8), xp[:, None],
        W1_l, W1_r, b1[None, :], W2_l, W2_r, b2[None, :])

    s2 = _sc_agg64(srcp, dstp, g1b.reshape(4 * NPAD, 16))

    z = _tc_dense2(s2.reshape(4, NPAD, 16), r1b, rec,
                   W_lin.reshape(4, 16, 4), b_lin[None, :])

    e0 = jnp.concatenate([train_pos_edge_index[0], negative_edge_index[0]]
                         ).astype(jnp.int32)
    e1 = jnp.concatenate([train_pos_edge_index[1], negative_edge_index[1]]
                         ).astype(jnp.int32)
    dpad = jnp.arange(EDPAD - ED, dtype=jnp.int32) % jnp.int32(N)
    e0p = jnp.concatenate([e0, dpad])
    e1p = jnp.concatenate([e1, dpad])

    out = _sc_decode(e0p, e1p, z)
    return out.reshape(EDPAD, 4)[:ED]


# speedup vs baseline: 9.5605x; 2.0490x over previous
import functools

import jax
import jax.numpy as jnp
from jax import lax
from jax.experimental import pallas as pl
from jax.experimental.pallas import tpu as pltpu
from jax.experimental.pallas import tpu_sc as plsc

N = 100000
E = 1600000
EP = 500000

NC, NS, L = 2, 16, 16
BN = 512
NPAD = 100352
ROWS_PER_TILE = NPAD // NS
C = 1024
CF = 128
EPAD = 1605632
HALF_E = EPAD // 2
CHUNKS1 = HALF_E // (NS * C)
CHUNKS2 = EPAD // (NS * C)
ED = 2 * EP
EDPAD = 1015808
DCHUNKS = EDPAD // (NC * NS * C)

_MESH = plsc.VectorSubcoreMesh(
    core_axis_name="c", subcore_axis_name="s", num_cores=NC, num_subcores=NS)
_SC_PARAMS = pltpu.CompilerParams(
    use_tc_tiling_on_sc=False, needs_layout_passes=False)


def _zero_rows8(ref, rows):
    iota = jnp.arange(L, dtype=jnp.int32)
    z = jnp.zeros((L,), jnp.float32)
    for g in range(rows // L):
        er = iota + (g * L)
        for col in range(8):
            plsc.store_scatter(ref, [er, jnp.full((L,), col, jnp.int32)], z)


@functools.partial(
    pl.kernel,
    out_type=jax.ShapeDtypeStruct((2 * NPAD, 8), jnp.float32),
    mesh=_MESH,
    compiler_params=_SC_PARAMS,
    scratch_types=dict(
        sidx_v=pltpu.VMEM((C,), jnp.int32),
        didx_v=pltpu.VMEM((C,), jnp.int32),
        rows_v=pltpu.VMEM((C, 8), jnp.float32),
        zrows_v=pltpu.VMEM((CF, 8), jnp.float32),
        fbuf_v=pltpu.VMEM((CF, 8), jnp.float32),
        acc_sh=pltpu.VMEM_SHARED((NPAD, 8), jnp.float32),
        sem=pltpu.SemaphoreType.DMA,
    ),
)
def _sc_agg_scalar(src_hbm, dst_hbm, xr_hbm, out_hbm,
                   sidx_v, didx_v, rows_v, zrows_v, fbuf_v, acc_sh, sem):
    c = lax.axis_index("c")
    s = lax.axis_index("s")

    _zero_rows8(zrows_v, CF)

    def zbody(j, carry):
        pltpu.sync_copy(zrows_v,
                        acc_sh.at[pl.ds(s * ROWS_PER_TILE + j * CF, CF), :])
        return carry
    lax.fori_loop(0, ROWS_PER_TILE // CF, zbody, 0)
    plsc.subcore_barrier()

    def body(j, carry):
        base = c * HALF_E + (s * CHUNKS1 + j) * C
        pltpu.sync_copy(src_hbm.at[pl.ds(base, C)], sidx_v)
        pltpu.async_copy(xr_hbm.at[sidx_v], rows_v, sem).wait()
        pltpu.sync_copy(dst_hbm.at[pl.ds(base, C)], didx_v)
        pltpu.sync_copy(rows_v, acc_sh.at[didx_v], add=True)
        return carry
    lax.fori_loop(0, CHUNKS1, body, 0)
    plsc.subcore_barrier()

    def fbody(j, carry):
        row = s * ROWS_PER_TILE + j * CF
        pltpu.sync_copy(acc_sh.at[pl.ds(row, CF), :], fbuf_v)
        pltpu.sync_copy(fbuf_v, out_hbm.at[pl.ds(c * NPAD + row, CF), :])
        return carry
    lax.fori_loop(0, ROWS_PER_TILE // CF, fbody, 0)


@functools.partial(
    pl.kernel,
    out_type=jax.ShapeDtypeStruct((4 * NPAD, 16), jnp.float32),
    mesh=_MESH,
    compiler_params=_SC_PARAMS,
    scratch_types=dict(
        sidx_v=pltpu.VMEM((C,), jnp.int32),
        didx_v=pltpu.VMEM((C,), jnp.int32),
        adj_v=pltpu.VMEM((C,), jnp.int32),
        rows_v=pltpu.VMEM((C, 16), jnp.float32),
        zrows_v=pltpu.VMEM((CF, 16), jnp.float32),
        fbuf_v=pltpu.VMEM((CF, 16), jnp.float32),
        acc_sh=pltpu.VMEM_SHARED((NPAD, 16), jnp.float32),
        sem=pltpu.SemaphoreType.DMA,
    ),
)
def _sc_agg64(src_hbm, dst_hbm, g1_hbm, s2_hbm,
              sidx_v, didx_v, adj_v, rows_v, zrows_v, fbuf_v, acc_sh, sem):
    c = lax.axis_index("c")
    s = lax.axis_index("s")

    for r in range(CF):
        zrows_v[r, :] = jnp.zeros((16,), jnp.float32)

    for rnd in range(2):
        fboff = (c * 2 + rnd) * NPAD

        def zbody(j, carry):
            pltpu.sync_copy(
                zrows_v, acc_sh.at[pl.ds(s * ROWS_PER_TILE + j * CF, CF), :])
            return carry
        lax.fori_loop(0, ROWS_PER_TILE // CF, zbody, 0)
        plsc.subcore_barrier()

        def body(j, carry):
            base = (s * CHUNKS2 + j) * C
            pltpu.sync_copy(src_hbm.at[pl.ds(base, C)], sidx_v)
            for k in range(C // L):
                sl = pl.ds(k * L, L)
                adj_v[sl] = sidx_v[sl] + fboff
            pltpu.async_copy(g1_hbm.at[adj_v], rows_v, sem).wait()
            pltpu.sync_copy(dst_hbm.at[pl.ds(base, C)], didx_v)
            pltpu.sync_copy(rows_v, acc_sh.at[didx_v], add=True)
            return carry
        lax.fori_loop(0, CHUNKS2, body, 0)
        plsc.subcore_barrier()

        def fbody(j, carry):
            row = s * ROWS_PER_TILE + j * CF
            pltpu.sync_copy(acc_sh.at[pl.ds(row, CF), :], fbuf_v)
            pltpu.sync_copy(fbuf_v, s2_hbm.at[pl.ds(fboff + row, CF), :])
            return carry
        lax.fori_loop(0, ROWS_PER_TILE // CF, fbody, 0)
        plsc.subcore_barrier()


@functools.partial(
    pl.kernel,
    out_type=jax.ShapeDtypeStruct((4 * EDPAD,), jnp.float32),
    mesh=_MESH,
    compiler_params=_SC_PARAMS,
    scratch_types=dict(
        i0_v=pltpu.VMEM((C,), jnp.int32),
        i1_v=pltpu.VMEM((C,), jnp.int32),
        r0_v=pltpu.VMEM((C, 16), jnp.float32),
        r1_v=pltpu.VMEM((C, 16), jnp.float32),
        o_v=pltpu.VMEM((4 * C,), jnp.float32),
        sem=pltpu.SemaphoreType.DMA,
    ),
)
def _sc_decode(e0_hbm, e1_hbm, z_hbm, out_hbm,
               i0_v, i1_v, r0_v, r1_v, o_v, sem):
    c = lax.axis_index("c")
    s = lax.axis_index("s")
    wid = s * NC + c
    iota = jnp.arange(L, dtype=jnp.int32)

    def body(j, carry):
        base = (wid * DCHUNKS + j) * C
        pltpu.sync_copy(e0_hbm.at[pl.ds(base, C)], i0_v)
        pltpu.sync_copy(e1_hbm.at[pl.ds(base, C)], i1_v)
        pltpu.async_copy(z_hbm.at[i0_v], r0_v, sem).wait()
        pltpu.async_copy(z_hbm.at[i1_v], r1_v, sem).wait()
        for g in range(C // L):
            er = iota + (g * L)
            lg = []
            for cls in range(4):
                csp = jnp.full((L,), cls, jnp.int32)
                v0 = plsc.load_gather(r0_v, [er, csp])
                v1 = plsc.load_gather(r1_v, [er, csp])
                lg.append(v0 * v1)
            m = jnp.maximum(jnp.maximum(lg[0], lg[1]),
                            jnp.maximum(lg[2], lg[3]))
            ex = [jnp.exp(v - m) for v in lg]
            tot = (ex[0] + ex[1]) + (ex[2] + ex[3])
            for cls in range(4):
                plsc.store_scatter(o_v, [er * 4 + cls], ex[cls] / tot)
        pltpu.sync_copy(o_v, out_hbm.at[pl.ds(base * 4, 4 * C)])
        return carry
    lax.fori_loop(0, DCHUNKS, body, 0)


def _tc_dense1_body(p_ref, x_ref, w1l_ref, w1r_ref, b1_ref,
                    w2l_ref, w2r_ref, b2_ref, g1_ref, r1_ref, rec_ref):
    s1 = p_ref[0, :, 0:1] + p_ref[1, :, 0:1]
    cnt = p_ref[0, :, 1:2] + p_ref[1, :, 1:2]
    rec = 1.0 / jnp.maximum(cnt, 1.0)
    rec_ref[...] = rec
    a = s1 * rec
    h1 = jnp.maximum(
        a * w1l_ref[...] + x_ref[...] * w1r_ref[...] + b1_ref[...], 0.0)
    g = jnp.dot(h1, w2l_ref[...], preferred_element_type=jnp.float32)
    r = jnp.dot(h1, w2r_ref[...], preferred_element_type=jnp.float32)
    r = r + b2_ref[...]
    for fb in range(4):
        g1_ref[fb, :, :] = g[:, fb * 16:(fb + 1) * 16]
        r1_ref[fb, :, :] = r[:, fb * 16:(fb + 1) * 16]


_tc_dense1 = pl.pallas_call(
    _tc_dense1_body,
    grid=(NPAD // BN,),
    in_specs=[
        pl.BlockSpec((2, BN, 8), lambda i: (0, i, 0)),
        pl.BlockSpec((BN, 1), lambda i: (i, 0)),
        pl.BlockSpec((1, 128), lambda i: (0, 0)),
        pl.BlockSpec((1, 128), lambda i: (0, 0)),
        pl.BlockSpec((1, 128), lambda i: (0, 0)),
        pl.BlockSpec((128, 64), lambda i: (0, 0)),
        pl.BlockSpec((128, 64), lambda i: (0, 0)),
        pl.BlockSpec((1, 64), lambda i: (0, 0)),
    ],
    out_specs=[
        pl.BlockSpec((4, BN, 16), lambda i: (0, i, 0)),
        pl.BlockSpec((4, BN, 16), lambda i: (0, i, 0)),
        pl.BlockSpec((BN, 1), lambda i: (i, 0)),
    ],
    out_shape=[
        jax.ShapeDtypeStruct((4, NPAD, 16), jnp.float32),
        jax.ShapeDtypeStruct((4, NPAD, 16), jnp.float32),
        jax.ShapeDtypeStruct((NPAD, 1), jnp.float32),
    ],
)


def _tc_dense2_body(s2_ref, r1_ref, rec_ref, wlin_ref, blin_ref, z_ref):
    rec = rec_ref[...]
    acc = jnp.zeros((BN, 4), jnp.float32)
    for fb in range(4):
        u = jnp.maximum(s2_ref[fb, :, :] * rec + r1_ref[fb, :, :], 0.0)
        acc = acc + jnp.dot(u, wlin_ref[fb, :, :],
                            preferred_element_type=jnp.float32)
    z_ref[...] = jnp.concatenate(
        [acc + blin_ref[...], jnp.zeros((BN, 12), jnp.float32)], axis=1)


_tc_dense2 = pl.pallas_call(
    _tc_dense2_body,
    grid=(NPAD // BN,),
    in_specs=[
        pl.BlockSpec((4, BN, 16), lambda i: (0, i, 0)),
        pl.BlockSpec((4, BN, 16), lambda i: (0, i, 0)),
        pl.BlockSpec((BN, 1), lambda i: (i, 0)),
        pl.BlockSpec((4, 16, 4), lambda i: (0, 0, 0)),
        pl.BlockSpec((1, 4), lambda i: (0, 0)),
    ],
    out_specs=pl.BlockSpec((BN, 16), lambda i: (i, 0)),
    out_shape=jax.ShapeDtypeStruct((NPAD, 16), jnp.float32),
)


def kernel(x, train_edge_index, train_pos_edge_index, negative_edge_index,
           W1_l, W1_r, b1, W2_l, W2_r, b2, W_lin, b_lin):
    src = train_edge_index[0].astype(jnp.int32)
    dst = train_edge_index[1].astype(jnp.int32)
    epad = jnp.int32(N) + (jnp.arange(EPAD - E, dtype=jnp.int32) % 256)
    srcp = jnp.concatenate([src, epad])
    dstp = jnp.concatenate([dst, epad])
    xp = jnp.pad(x[:, 0], (0, NPAD - N))
    xr = jnp.concatenate(
        [xp[:, None], jnp.ones((NPAD, 1), jnp.float32),
         jnp.zeros((NPAD, 6), jnp.float32)], axis=1)

    parts = _sc_agg_scalar(srcp, dstp, xr)

    g1b, r1b, rec = _tc_dense1(
        parts.reshape(2, NPAD, 8), xp[:, None],
        W1_l, W1_r, b1[None, :], W2_l, W2_r, b2[None, :])

    s2 = _sc_agg64(srcp, dstp, g1b.reshape(4 * NPAD, 16))

    z = _tc_dense2(s2.reshape(4, NPAD, 16), r1b, rec,
                   W_lin.reshape(4, 16, 4), b_lin[None, :])

    e0 = jnp.concatenate([train_pos_edge_index[0], negative_edge_index[0]]
                         ).astype(jnp.int32)
    e1 = jnp.concatenate([train_pos_edge_index[1], negative_edge_index[1]]
                         ).astype(jnp.int32)
    dpad = jnp.arange(EDPAD - ED, dtype=jnp.int32) % jnp.int32(N)
    e0p = jnp.concatenate([e0, dpad])
    e1p = jnp.concatenate([e1, dpad])

    out = _sc_decode(e0p, e1p, z)
    return out.reshape(EDPAD, 4)[:ED]


# speedup vs baseline: 10.5215x; 1.1005x over previous
import functools

import jax
import jax.numpy as jnp
from jax import lax
from jax.experimental import pallas as pl
from jax.experimental.pallas import tpu as pltpu
from jax.experimental.pallas import tpu_sc as plsc

N = 100000
E = 1600000
EP = 500000

NC, NS, L = 2, 16, 16
BN = 512
NPAD = 100352
ROWS_PER_TILE = NPAD // NS
C = 512
CD = 1024
CF = 128
EPAD = 1605632
HALF_E = EPAD // 2
CHUNKS1 = HALF_E // (NS * C)
CHUNKS2 = EPAD // (NS * C)
ED = 2 * EP
EDPAD = 1048576
DCHUNKS = EDPAD // (NC * NS * CD)

_MESH = plsc.VectorSubcoreMesh(
    core_axis_name="c", subcore_axis_name="s", num_cores=NC, num_subcores=NS)
_SC_PARAMS = pltpu.CompilerParams(
    use_tc_tiling_on_sc=False, needs_layout_passes=False)


def _zero_rows8(ref, rows):
    iota = jnp.arange(L, dtype=jnp.int32)
    z = jnp.zeros((L,), jnp.float32)
    for g in range(rows // L):
        er = iota + (g * L)
        for col in range(8):
            plsc.store_scatter(ref, [er, jnp.full((L,), col, jnp.int32)], z)


def _agg_pipeline(src_hbm, dst_hbm, tbl_hbm, acc_sh, bufs, chunk0, nchunks):
    sidx0, didx0, rows0, sem0, sidx1, didx1, rows1, sem1 = bufs

    def load_idx(j, sidx, didx):
        base = (chunk0 + j) * C
        pltpu.sync_copy(src_hbm.at[pl.ds(base, C)], sidx)
        pltpu.sync_copy(dst_hbm.at[pl.ds(base, C)], didx)

    load_idx(0, sidx0, didx0)
    pltpu.async_copy(tbl_hbm.at[sidx0], rows0, sem0)

    def body(i, carry):
        a = 2 * i
        load_idx(a + 1, sidx1, didx1)
        pltpu.async_copy(tbl_hbm.at[sidx1], rows1, sem1)
        pltpu.make_async_copy(tbl_hbm.at[sidx0], rows0, sem0).wait()
        pltpu.sync_copy(rows0, acc_sh.at[didx0], add=True)
        load_idx(a + 2, sidx0, didx0)
        pltpu.async_copy(tbl_hbm.at[sidx0], rows0, sem0)
        pltpu.make_async_copy(tbl_hbm.at[sidx1], rows1, sem1).wait()
        pltpu.sync_copy(rows1, acc_sh.at[didx1], add=True)
        return carry
    lax.fori_loop(0, nchunks // 2 - 1, body, 0)

    load_idx(nchunks - 1, sidx1, didx1)
    pltpu.async_copy(tbl_hbm.at[sidx1], rows1, sem1)
    pltpu.make_async_copy(tbl_hbm.at[sidx0], rows0, sem0).wait()
    pltpu.sync_copy(rows0, acc_sh.at[didx0], add=True)
    pltpu.make_async_copy(tbl_hbm.at[sidx1], rows1, sem1).wait()
    pltpu.sync_copy(rows1, acc_sh.at[didx1], add=True)


@functools.partial(
    pl.kernel,
    out_type=jax.ShapeDtypeStruct((2 * NPAD, 8), jnp.float32),
    mesh=_MESH,
    compiler_params=_SC_PARAMS,
    scratch_types=dict(
        sidx0=pltpu.VMEM((C,), jnp.int32),
        didx0=pltpu.VMEM((C,), jnp.int32),
        rows0=pltpu.VMEM((C, 8), jnp.float32),
        sidx1=pltpu.VMEM((C,), jnp.int32),
        didx1=pltpu.VMEM((C,), jnp.int32),
        rows1=pltpu.VMEM((C, 8), jnp.float32),
        zrows_v=pltpu.VMEM((CF, 8), jnp.float32),
        fbuf_v=pltpu.VMEM((CF, 8), jnp.float32),
        acc_sh=pltpu.VMEM_SHARED((NPAD, 8), jnp.float32),
        sem0=pltpu.SemaphoreType.DMA,
        sem1=pltpu.SemaphoreType.DMA,
    ),
)
def _sc_agg_scalar(src_hbm, dst_hbm, xr_hbm, out_hbm,
                   sidx0, didx0, rows0, sidx1, didx1, rows1,
                   zrows_v, fbuf_v, acc_sh, sem0, sem1):
    c = lax.axis_index("c")
    s = lax.axis_index("s")

    _zero_rows8(zrows_v, CF)

    def zbody(j, carry):
        pltpu.sync_copy(zrows_v,
                        acc_sh.at[pl.ds(s * ROWS_PER_TILE + j * CF, CF), :])
        return carry
    lax.fori_loop(0, ROWS_PER_TILE // CF, zbody, 0)
    plsc.subcore_barrier()

    chunk0 = c * (HALF_E // C) + s * CHUNKS1
    _agg_pipeline(src_hbm, dst_hbm, xr_hbm, acc_sh,
                  (sidx0, didx0, rows0, sem0, sidx1, didx1, rows1, sem1),
                  chunk0, CHUNKS1)
    plsc.subcore_barrier()

    def fbody(j, carry):
        row = s * ROWS_PER_TILE + j * CF
        pltpu.sync_copy(acc_sh.at[pl.ds(row, CF), :], fbuf_v)
        pltpu.sync_copy(fbuf_v, out_hbm.at[pl.ds(c * NPAD + row, CF), :])
        return carry
    lax.fori_loop(0, ROWS_PER_TILE // CF, fbody, 0)


@functools.partial(
    pl.kernel,
    out_type=tuple(jax.ShapeDtypeStruct((NPAD, 16), jnp.float32)
                   for _ in range(4)),
    mesh=_MESH,
    compiler_params=_SC_PARAMS,
    scratch_types=dict(
        sidx0=pltpu.VMEM((C,), jnp.int32),
        didx0=pltpu.VMEM((C,), jnp.int32),
        rows0=pltpu.VMEM((C, 16), jnp.float32),
        sidx1=pltpu.VMEM((C,), jnp.int32),
        didx1=pltpu.VMEM((C,), jnp.int32),
        rows1=pltpu.VMEM((C, 16), jnp.float32),
        zrows_v=pltpu.VMEM((CF, 16), jnp.float32),
        fbuf_v=pltpu.VMEM((CF, 16), jnp.float32),
        acc_sh=pltpu.VMEM_SHARED((NPAD, 16), jnp.float32),
        sem0=pltpu.SemaphoreType.DMA,
        sem1=pltpu.SemaphoreType.DMA,
    ),
)
def _sc_agg64(src_hbm, dst_hbm, g1_0, g1_1, g1_2, g1_3,
              s2_0, s2_1, s2_2, s2_3,
              sidx0, didx0, rows0, sidx1, didx1, rows1,
              zrows_v, fbuf_v, acc_sh, sem0, sem1):
    c = lax.axis_index("c")
    s = lax.axis_index("s")

    for r in range(CF):
        zrows_v[r, :] = jnp.zeros((16,), jnp.float32)

    def round_body(g1_hbm, out_hbm):
        def zbody(j, carry):
            pltpu.sync_copy(
                zrows_v, acc_sh.at[pl.ds(s * ROWS_PER_TILE + j * CF, CF), :])
            return carry
        lax.fori_loop(0, ROWS_PER_TILE // CF, zbody, 0)
        plsc.subcore_barrier()
        _agg_pipeline(src_hbm, dst_hbm, g1_hbm, acc_sh,
                      (sidx0, didx0, rows0, sem0, sidx1, didx1, rows1, sem1),
                      s * CHUNKS2, CHUNKS2)
        plsc.subcore_barrier()

        def fbody(j, carry):
            row = s * ROWS_PER_TILE + j * CF
            pltpu.sync_copy(acc_sh.at[pl.ds(row, CF), :], fbuf_v)
            pltpu.sync_copy(fbuf_v, out_hbm.at[pl.ds(row, CF), :])
            return carry
        lax.fori_loop(0, ROWS_PER_TILE // CF, fbody, 0)
        plsc.subcore_barrier()

    for rnd in range(2):
        @pl.when(c == 0)
        def _core0():
            round_body((g1_0, g1_1)[rnd], (s2_0, s2_1)[rnd])

        @pl.when(c == 1)
        def _core1():
            round_body((g1_2, g1_3)[rnd], (s2_2, s2_3)[rnd])


@functools.partial(
    pl.kernel,
    out_type=jax.ShapeDtypeStruct((4 * EDPAD,), jnp.float32),
    mesh=_MESH,
    compiler_params=_SC_PARAMS,
    scratch_types=dict(
        i0a=pltpu.VMEM((CD,), jnp.int32),
        i1a=pltpu.VMEM((CD,), jnp.int32),
        r0a=pltpu.VMEM((CD, 16), jnp.float32),
        r1a=pltpu.VMEM((CD, 16), jnp.float32),
        oa=pltpu.VMEM((4 * CD,), jnp.float32),
        i0b=pltpu.VMEM((CD,), jnp.int32),
        i1b=pltpu.VMEM((CD,), jnp.int32),
        r0b=pltpu.VMEM((CD, 16), jnp.float32),
        r1b=pltpu.VMEM((CD, 16), jnp.float32),
        ob=pltpu.VMEM((4 * CD,), jnp.float32),
        sema=pltpu.SemaphoreType.DMA,
        semb=pltpu.SemaphoreType.DMA,
    ),
)
def _sc_decode(e0_hbm, e1_hbm, z_hbm, out_hbm,
               i0a, i1a, r0a, r1a, oa, i0b, i1b, r0b, r1b, ob, sema, semb):
    c = lax.axis_index("c")
    s = lax.axis_index("s")
    wid = s * NC + c
    iota = jnp.arange(L, dtype=jnp.int32)

    def start(j, i0, i1, r0, r1, sem):
        base = (wid * DCHUNKS + j) * CD
        pltpu.sync_copy(e0_hbm.at[pl.ds(base, CD)], i0)
        pltpu.sync_copy(e1_hbm.at[pl.ds(base, CD)], i1)
        pltpu.async_copy(z_hbm.at[i0], r0, sem)
        pltpu.async_copy(z_hbm.at[i1], r1, sem)

    def finish(j, i0, i1, r0_v, r1_v, o_v, sem):
        pltpu.make_async_copy(z_hbm.at[i0], r0_v, sem).wait()
        pltpu.make_async_copy(z_hbm.at[i1], r1_v, sem).wait()

        def gbody(g, carry):
            er = iota + g * L
            lg = []
            for cls in range(4):
                csp = jnp.full((L,), cls, jnp.int32)
                v0 = plsc.load_gather(r0_v, [er, csp])
                v1 = plsc.load_gather(r1_v, [er, csp])
                lg.append(v0 * v1)
            m = jnp.maximum(jnp.maximum(lg[0], lg[1]),
                            jnp.maximum(lg[2], lg[3]))
            ex = [jnp.exp(v - m) for v in lg]
            tot = (ex[0] + ex[1]) + (ex[2] + ex[3])
            for cls in range(4):
                plsc.store_scatter(o_v, [er * 4 + cls], ex[cls] / tot)
            return carry
        lax.fori_loop(0, CD // L, gbody, 0)
        base = (wid * DCHUNKS + j) * CD
        pltpu.sync_copy(o_v, out_hbm.at[pl.ds(base * 4, 4 * CD)])

    start(0, i0a, i1a, r0a, r1a, sema)

    def body(i, carry):
        a = 2 * i
        start(a + 1, i0b, i1b, r0b, r1b, semb)
        finish(a, i0a, i1a, r0a, r1a, oa, sema)
        start(a + 2, i0a, i1a, r0a, r1a, sema)
        finish(a + 1, i0b, i1b, r0b, r1b, ob, semb)
        return carry
    lax.fori_loop(0, DCHUNKS // 2 - 1, body, 0)

    start(DCHUNKS - 1, i0b, i1b, r0b, r1b, semb)
    finish(DCHUNKS - 2, i0a, i1a, r0a, r1a, oa, sema)
    finish(DCHUNKS - 1, i0b, i1b, r0b, r1b, ob, semb)


def _tc_dense1_body(p_ref, x_ref, w1l_ref, w1r_ref, b1_ref,
                    w2l_ref, w2r_ref, b2_ref,
                    g10_ref, g11_ref, g12_ref, g13_ref, r1_ref, rec_ref):
    s1 = p_ref[0, :, 0:1] + p_ref[1, :, 0:1]
    cnt = p_ref[0, :, 1:2] + p_ref[1, :, 1:2]
    rec = 1.0 / jnp.maximum(cnt, 1.0)
    rec_ref[...] = rec
    a = s1 * rec
    h1 = jnp.maximum(
        a * w1l_ref[...] + x_ref[...] * w1r_ref[...] + b1_ref[...], 0.0)
    g = jnp.dot(h1, w2l_ref[...], preferred_element_type=jnp.float32)
    r = jnp.dot(h1, w2r_ref[...], preferred_element_type=jnp.float32)
    r = r + b2_ref[...]
    for fb, gref in enumerate([g10_ref, g11_ref, g12_ref, g13_ref]):
        gref[...] = g[:, fb * 16:(fb + 1) * 16]
        r1_ref[fb, :, :] = r[:, fb * 16:(fb + 1) * 16]


_tc_dense1 = pl.pallas_call(
    _tc_dense1_body,
    grid=(NPAD // BN,),
    in_specs=[
        pl.BlockSpec((2, BN, 8), lambda i: (0, i, 0)),
        pl.BlockSpec((BN, 1), lambda i: (i, 0)),
        pl.BlockSpec((1, 128), lambda i: (0, 0)),
        pl.BlockSpec((1, 128), lambda i: (0, 0)),
        pl.BlockSpec((1, 128), lambda i: (0, 0)),
        pl.BlockSpec((128, 64), lambda i: (0, 0)),
        pl.BlockSpec((128, 64), lambda i: (0, 0)),
        pl.BlockSpec((1, 64), lambda i: (0, 0)),
    ],
    out_specs=[
        pl.BlockSpec((BN, 16), lambda i: (i, 0)),
        pl.BlockSpec((BN, 16), lambda i: (i, 0)),
        pl.BlockSpec((BN, 16), lambda i: (i, 0)),
        pl.BlockSpec((BN, 16), lambda i: (i, 0)),
        pl.BlockSpec((4, BN, 16), lambda i: (0, i, 0)),
        pl.BlockSpec((BN, 1), lambda i: (i, 0)),
    ],
    out_shape=[
        jax.ShapeDtypeStruct((NPAD, 16), jnp.float32),
        jax.ShapeDtypeStruct((NPAD, 16), jnp.float32),
        jax.ShapeDtypeStruct((NPAD, 16), jnp.float32),
        jax.ShapeDtypeStruct((NPAD, 16), jnp.float32),
        jax.ShapeDtypeStruct((4, NPAD, 16), jnp.float32),
        jax.ShapeDtypeStruct((NPAD, 1), jnp.float32),
    ],
)


def _tc_dense2_body(s20_ref, s21_ref, s22_ref, s23_ref, r1_ref, rec_ref,
                    wlin_ref, blin_ref, z_ref):
    rec = rec_ref[...]
    acc = jnp.zeros((BN, 4), jnp.float32)
    for fb, sref in enumerate([s20_ref, s21_ref, s22_ref, s23_ref]):
        u = jnp.maximum(sref[...] * rec + r1_ref[fb, :, :], 0.0)
        acc = acc + jnp.dot(u, wlin_ref[fb, :, :],
                            preferred_element_type=jnp.float32)
    z_ref[...] = jnp.concatenate(
        [acc + blin_ref[...], jnp.zeros((BN, 12), jnp.float32)], axis=1)


_tc_dense2 = pl.pallas_call(
    _tc_dense2_body,
    grid=(NPAD // BN,),
    in_specs=[
        pl.BlockSpec((BN, 16), lambda i: (i, 0)),
        pl.BlockSpec((BN, 16), lambda i: (i, 0)),
        pl.BlockSpec((BN, 16), lambda i: (i, 0)),
        pl.BlockSpec((BN, 16), lambda i: (i, 0)),
        pl.BlockSpec((4, BN, 16), lambda i: (0, i, 0)),
        pl.BlockSpec((BN, 1), lambda i: (i, 0)),
        pl.BlockSpec((4, 16, 4), lambda i: (0, 0, 0)),
        pl.BlockSpec((1, 4), lambda i: (0, 0)),
    ],
    out_specs=pl.BlockSpec((BN, 16), lambda i: (i, 0)),
    out_shape=jax.ShapeDtypeStruct((NPAD, 16), jnp.float32),
)


def kernel(x, train_edge_index, train_pos_edge_index, negative_edge_index,
           W1_l, W1_r, b1, W2_l, W2_r, b2, W_lin, b_lin):
    src = train_edge_index[0].astype(jnp.int32)
    dst = train_edge_index[1].astype(jnp.int32)
    epad = jnp.int32(N) + (jnp.arange(EPAD - E, dtype=jnp.int32) % 256)
    srcp = jnp.concatenate([src, epad])
    dstp = jnp.concatenate([dst, epad])
    xp = jnp.pad(x[:, 0], (0, NPAD - N))
    xr = jnp.concatenate(
        [xp[:, None], jnp.ones((NPAD, 1), jnp.float32),
         jnp.zeros((NPAD, 6), jnp.float32)], axis=1)

    parts = _sc_agg_scalar(srcp, dstp, xr)

    g10, g11, g12, g13, r1b, rec = _tc_dense1(
        parts.reshape(2, NPAD, 8), xp[:, None],
        W1_l, W1_r, b1[None, :], W2_l, W2_r, b2[None, :])

    s20, s21, s22, s23 = _sc_agg64(srcp, dstp, g10, g11, g12, g13)

    z = _tc_dense2(s20, s21, s22, s23, r1b, rec,
                   W_lin.reshape(4, 16, 4), b_lin[None, :])

    e0 = jnp.concatenate([train_pos_edge_index[0], negative_edge_index[0]]
                         ).astype(jnp.int32)
    e1 = jnp.concatenate([train_pos_edge_index[1], negative_edge_index[1]]
                         ).astype(jnp.int32)
    dpad = jnp.arange(EDPAD - ED, dtype=jnp.int32) % jnp.int32(N)
    e0p = jnp.concatenate([e0, dpad])
    e1p = jnp.concatenate([e1, dpad])

    out = _sc_decode(e0p, e1p, z)
    return out.reshape(EDPAD, 4)[:ED]
